# baseline (speedup 1.0000x reference)
import functools
import jax
import jax.numpy as jnp
from jax import lax
from jax.experimental import pallas as pl
from jax.experimental.pallas import tpu as pltpu
from jax.experimental.pallas import tpu_sc as plsc

D = 64
NC, NS, L = 2, 16, 16
NW = NC * NS
SCALE = 8.0
NBUF = 2


def _make_kernel(BATCH, SEQ):
    B = BATCH * SEQ
    b_per_w = B // NW
    p_per_w = BATCH // NW
    n_steps = p_per_w
    assert n_steps % NBUF == 0
    n_rounds = n_steps // NBUF
    mesh = plsc.VectorSubcoreMesh(
        core_axis_name="c", subcore_axis_name="s",
        num_cores=NC, num_subcores=NS,
    )

    scratch = dict(
        idx_all=pltpu.VMEM((b_per_w,), jnp.int32),
        gbuf=[pltpu.VMEM((SEQ, 2 * D), jnp.float32) for _ in range(NBUF)],
        sbuf=[pltpu.VMEM((SEQ, D), jnp.float32) for _ in range(NBUF)],
        gsem=[pltpu.SemaphoreType.DMA for _ in range(NBUF)],
        ssem=[pltpu.SemaphoreType.DMA for _ in range(NBUF)],
    )

    @functools.partial(
        pl.kernel,
        mesh=mesh,
        compiler_params=pltpu.CompilerParams(use_tc_tiling_on_sc=True),
        out_type=jax.ShapeDtypeStruct((BATCH, SEQ, D), jnp.float32),
        scratch_types=scratch,
    )
    def k(x_hbm, table_hbm, out_hbm, idx_all, gbuf, sbuf, gsem, ssem):
        wid = lax.axis_index("s") * NC + lax.axis_index("c")
        rbase = wid * b_per_w
        pbase = wid * p_per_w

        pltpu.sync_copy(x_hbm.at[pl.ds(rbase, b_per_w)], idx_all)

        def issue_gather(c, b):
            pltpu.async_copy(
                table_hbm.at[idx_all.at[pl.ds(c * SEQ, SEQ)]],
                gbuf[b], gsem[b])

        def issue_store(c, b):
            pltpu.async_copy(
                sbuf[b], out_hbm.at[pbase + c], ssem[b])

        def wait_gather(b):
            pltpu.make_async_copy(table_hbm.at[idx_all.at[pl.ds(0, SEQ)]],
                                  gbuf[b], gsem[b]).wait()

        def wait_store(b):
            pltpu.make_async_copy(sbuf[b], out_hbm.at[0], ssem[b]).wait()

        def scale(b):
            @plsc.parallel_loop(0, SEQ, unroll=8)
            def row_body(s):
                for j in range(D // L):
                    sl = pl.ds(j * L, L)
                    sbuf[b][s, sl] = gbuf[b][s, sl] * SCALE

        for b in range(NBUF):
            issue_gather(b, b)

        for b in range(NBUF):
            wait_gather(b)
            scale(b)
            issue_gather(NBUF + b, b)
            issue_store(b, b)

        def round_body(r, carry):
            c0 = r * NBUF
            for b in range(NBUF):
                c = c0 + b
                wait_gather(b)
                wait_store(b)
                scale(b)
                issue_gather(c + NBUF, b)
                issue_store(c, b)
            return carry
        lax.fori_loop(1, n_rounds - 1, round_body, 0)

        for b in range(NBUF):
            c = (n_rounds - 1) * NBUF + b
            wait_gather(b)
            wait_store(b)
            scale(b)
            issue_store(c, b)
        for b in range(NBUF):
            wait_store(b)

    return k


def kernel(x, table):
    BATCH, SEQ = x.shape
    xf = x.reshape(BATCH * SEQ).astype(jnp.int32)
    table2 = jnp.concatenate([table, table], axis=1)
    return _make_kernel(BATCH, SEQ)(xf, table2)


# speedup vs baseline: 1.1390x; 1.1390x over previous
import functools
import jax
import jax.numpy as jnp
from jax import lax
from jax.experimental import pallas as pl
from jax.experimental.pallas import tpu as pltpu
from jax.experimental.pallas import tpu_sc as plsc

D = 64
NC, NS, L = 2, 16, 16
NW = NC * NS
SCALE = 8.0
NBUF = 2


def _make_kernel(BATCH, SEQ):
    p_per_w = BATCH // NW
    n_steps = p_per_w
    assert n_steps % NBUF == 0
    n_rounds = n_steps // NBUF
    mesh = plsc.VectorSubcoreMesh(
        core_axis_name="c", subcore_axis_name="s",
        num_cores=NC, num_subcores=NS,
    )

    scratch = dict(
        ibuf=[pltpu.VMEM((SEQ,), jnp.int32) for _ in range(NBUF)],
        gbuf=[pltpu.VMEM((SEQ, 2 * D), jnp.float32) for _ in range(NBUF)],
        sbuf=[pltpu.VMEM((SEQ, D), jnp.float32) for _ in range(NBUF)],
        isem=[pltpu.SemaphoreType.DMA for _ in range(NBUF)],
        gsem=[pltpu.SemaphoreType.DMA for _ in range(NBUF)],
        ssem=[pltpu.SemaphoreType.DMA for _ in range(NBUF)],
    )

    @functools.partial(
        pl.kernel,
        mesh=mesh,
        compiler_params=pltpu.CompilerParams(use_tc_tiling_on_sc=True),
        out_type=jax.ShapeDtypeStruct((BATCH, SEQ, D), jnp.float32),
        scratch_types=scratch,
    )
    def k(x_hbm, table_hbm, out_hbm, ibuf, gbuf, sbuf, isem, gsem, ssem):
        wid = lax.axis_index("s") * NC + lax.axis_index("c")
        pbase = wid * p_per_w

        def issue_idx(c, b):
            pltpu.async_copy(x_hbm.at[pbase + c], ibuf[b], isem[b])

        def wait_idx(b):
            pltpu.make_async_copy(x_hbm.at[0], ibuf[b], isem[b]).wait()

        def issue_gather(b):
            pltpu.async_copy(table_hbm.at[ibuf[b]], gbuf[b], gsem[b])

        def wait_gather(b):
            pltpu.make_async_copy(table_hbm.at[ibuf[b]], gbuf[b],
                                  gsem[b]).wait()

        def issue_store(c, b):
            pltpu.async_copy(sbuf[b], out_hbm.at[pbase + c], ssem[b])

        def wait_store(b):
            pltpu.make_async_copy(sbuf[b], out_hbm.at[0], ssem[b]).wait()

        def scale(b):
            @plsc.parallel_loop(0, SEQ, unroll=8)
            def row_body(s):
                for j in range(D // L):
                    sl = pl.ds(j * L, L)
                    sbuf[b][s, sl] = gbuf[b][s, sl] * SCALE

        for b in range(NBUF):
            issue_idx(b, b)
        for b in range(NBUF):
            wait_idx(b)
            issue_gather(b)

        for b in range(NBUF):
            wait_gather(b)
            issue_idx(NBUF + b, b)
            scale(b)
            issue_store(b, b)
            wait_idx(b)
            issue_gather(b)

        def round_body(r, carry):
            c0 = r * NBUF
            for b in range(NBUF):
                c = c0 + b
                wait_gather(b)
                issue_idx(c + NBUF, b)
                wait_store(b)
                scale(b)
                issue_store(c, b)
                wait_idx(b)
                issue_gather(b)
            return carry
        lax.fori_loop(1, n_rounds - 1, round_body, 0)

        for b in range(NBUF):
            wait_gather(b)
            wait_store(b)
            scale(b)
            issue_store((n_rounds - 1) * NBUF + b, b)
        for b in range(NBUF):
            wait_store(b)

    return k


def kernel(x, table):
    BATCH, SEQ = x.shape
    table2 = jnp.pad(table, ((0, 0), (0, D)))
    return _make_kernel(BATCH, SEQ)(x.astype(jnp.int32), table2)
